# Initial kernel scaffold; baseline (speedup 1.0000x reference)
import jax
import jax.numpy as jnp
from jax.experimental import pallas as pl


def kernel(x, edge_index, edge_weight, W1, b1, W2, b2, W3, b3, W4, b4):
    raise NotImplementedError("write your pallas kernel here")


# speedup vs baseline: 13.0688x; 13.0688x over previous
import functools

import jax
import jax.numpy as jnp
from jax import lax
from jax.experimental import pallas as pl
from jax.experimental.pallas import tpu as pltpu
from jax.experimental.pallas import tpu_sc as plsc

NC = 2
NS = 16
L = 16
NT = NC * NS
EC = 128


def _round_up(v, m):
    return (v + m - 1) // m * m


def _sc_mesh():
    return plsc.VectorSubcoreMesh(core_axis_name="c", subcore_axis_name="s",
                                  num_cores=NC, num_subcores=NS)


def _wid():
    return lax.axis_index("s") * NC + lax.axis_index("c")


def _deg_kernel(ep, npad):
    chunks = ep // (NT * EC)
    rpt = npad // NS
    zq = rpt // EC

    @functools.partial(
        pl.kernel,
        out_type=jax.ShapeDtypeStruct((NC, npad), jnp.float32),
        mesh=_sc_mesh(),
        scratch_types=[
            pltpu.VMEM((EC,), jnp.int32),
            pltpu.VMEM((EC,), jnp.float32),
            pltpu.VMEM((EC,), jnp.float32),
            pltpu.VMEM_SHARED((npad,), jnp.float32),
            pltpu.SemaphoreType.DMA,
        ],
    )
    def k(dst_hbm, ew_hbm, out_hbm, idx_v, ew_v, zero_v, acc, sem):
        c = lax.axis_index("c")
        s = lax.axis_index("s")
        wid = s * NC + c
        for t in range(EC // L):
            zero_v[pl.ds(t * L, L)] = jnp.zeros((L,), jnp.float32)
        for q in range(zq):
            pltpu.sync_copy(zero_v, acc.at[pl.ds(s * rpt + q * EC, EC)])
        plsc.subcore_barrier()

        def body(i, _):
            base = pl.multiple_of(wid * (chunks * EC) + i * EC, EC)
            pltpu.sync_copy(dst_hbm.at[pl.ds(base, EC)], idx_v)
            pltpu.sync_copy(ew_hbm.at[pl.ds(base, EC)], ew_v)
            pltpu.sync_copy(ew_v, acc.at[idx_v], add=True)
            return 0

        lax.fori_loop(0, chunks, body, 0)
        plsc.subcore_barrier()
        pltpu.sync_copy(acc.at[pl.ds(s * rpt, rpt)],
                        out_hbm.at[c, pl.ds(s * rpt, rpt)])

    return k


def _norm_kernel(ep):
    chunks = ep // (NT * EC)

    @functools.partial(
        pl.kernel,
        out_type=jax.ShapeDtypeStruct((ep,), jnp.float32),
        mesh=_sc_mesh(),
        scratch_types=[
            pltpu.VMEM((EC,), jnp.int32),
            pltpu.VMEM((EC,), jnp.int32),
            pltpu.VMEM((EC,), jnp.float32),
            pltpu.VMEM((EC,), jnp.float32),
            pltpu.VMEM((EC,), jnp.float32),
            pltpu.SemaphoreType.DMA,
        ],
    )
    def k(src_hbm, dst_hbm, ew_hbm, dinv_hbm, out_hbm,
          src_v, dst_v, ew_v, g1_v, g2_v, sem):
        c = lax.axis_index("c")
        s = lax.axis_index("s")
        wid = s * NC + c

        def body(i, _):
            base = pl.multiple_of(wid * (chunks * EC) + i * EC, EC)
            pltpu.sync_copy(src_hbm.at[pl.ds(base, EC)], src_v)
            pltpu.sync_copy(dst_hbm.at[pl.ds(base, EC)], dst_v)
            pltpu.sync_copy(ew_hbm.at[pl.ds(base, EC)], ew_v)
            pltpu.async_copy(dinv_hbm.at[src_v], g1_v, sem).wait()
            pltpu.async_copy(dinv_hbm.at[dst_v], g2_v, sem).wait()
            for t in range(EC // L):
                sl = pl.ds(t * L, L)
                ew_v[sl] = g1_v[sl] * ew_v[sl] * g2_v[sl]
            pltpu.sync_copy(ew_v, out_hbm.at[pl.ds(base, EC)])
            return 0

        lax.fori_loop(0, chunks, body, 0)

    return k


def _spmm_kernel(ep, npad, nb):
    chunks = ep // (NT * EC)
    rpt = npad // NS
    zq = rpt // EC

    @functools.partial(
        pl.kernel,
        out_type=jax.ShapeDtypeStruct((NC, nb, npad, 128), jnp.float32),
        mesh=_sc_mesh(),
        scratch_types=[
            pltpu.VMEM((EC,), jnp.int32),
            pltpu.VMEM((EC,), jnp.int32),
            pltpu.VMEM((EC,), jnp.int32),
            pltpu.VMEM((EC,), jnp.float32),
            pltpu.VMEM((EC, 128), jnp.float32),
            pltpu.VMEM((EC, 128), jnp.float32),
            pltpu.VMEM_SHARED((npad, 128), jnp.float32),
            pltpu.SemaphoreType.DMA,
        ],
    )
    def k(hw_hbm, src_hbm, dst_hbm, norm_hbm, out_hbm,
          src_v, idxb_v, dst_v, norm_v, rows_v, zero_v, acc, sem):
        c = lax.axis_index("c")
        s = lax.axis_index("s")
        wid = s * NC + c

        def zbody(j, _):
            for t in range(128 // L):
                zero_v[j, pl.ds(t * L, L)] = jnp.zeros((L,), jnp.float32)
            return 0

        lax.fori_loop(0, EC, zbody, 0)

        for b in range(nb):
            for q in range(zq):
                pltpu.sync_copy(zero_v, acc.at[pl.ds(s * rpt + q * EC, EC)])
            plsc.subcore_barrier()

            def body(i, _):
                base = pl.multiple_of(wid * (chunks * EC) + i * EC, EC)
                pltpu.sync_copy(src_hbm.at[pl.ds(base, EC)], src_v)
                pltpu.sync_copy(dst_hbm.at[pl.ds(base, EC)], dst_v)
                pltpu.sync_copy(norm_hbm.at[pl.ds(base, EC)], norm_v)
                for t in range(EC // L):
                    sl = pl.ds(t * L, L)
                    idxb_v[sl] = src_v[sl] + b * npad
                pltpu.async_copy(hw_hbm.at[idxb_v], rows_v, sem).wait()
                for g in range(EC // L):
                    n16 = norm_v[pl.ds(g * L, L)]
                    for j in range(L):
                        nj = n16[j]
                        for t in range(128 // L):
                            sl = pl.ds(t * L, L)
                            e = g * L + j
                            rows_v[e, sl] = rows_v[e, sl] * nj
                pltpu.sync_copy(rows_v, acc.at[dst_v], add=True)
                return 0

            lax.fori_loop(0, chunks, body, 0)
            plsc.subcore_barrier()
            pltpu.sync_copy(acc.at[pl.ds(s * rpt, rpt)],
                            out_hbm.at[c, b, pl.ds(s * rpt, rpt)])
            plsc.subcore_barrier()

    return k


def _spmm1_kernel(ep, npad, nb):
    chunks = ep // (NT * EC)
    rpt = npad // NS
    zq = rpt // EC

    @functools.partial(
        pl.kernel,
        out_type=jax.ShapeDtypeStruct((NC, nb, npad), jnp.float32),
        mesh=_sc_mesh(),
        scratch_types=[
            pltpu.VMEM((EC,), jnp.int32),
            pltpu.VMEM((EC,), jnp.int32),
            pltpu.VMEM((EC,), jnp.int32),
            pltpu.VMEM((EC,), jnp.float32),
            pltpu.VMEM((EC,), jnp.float32),
            pltpu.VMEM((EC,), jnp.float32),
            pltpu.VMEM_SHARED((npad,), jnp.float32),
            pltpu.SemaphoreType.DMA,
        ],
    )
    def k(hw_hbm, src_hbm, dst_hbm, norm_hbm, out_hbm,
          src_v, idxb_v, dst_v, norm_v, rows_v, zero_v, acc, sem):
        c = lax.axis_index("c")
        s = lax.axis_index("s")
        wid = s * NC + c
        for t in range(EC // L):
            zero_v[pl.ds(t * L, L)] = jnp.zeros((L,), jnp.float32)

        for b in range(nb):
            for q in range(zq):
                pltpu.sync_copy(zero_v, acc.at[pl.ds(s * rpt + q * EC, EC)])
            plsc.subcore_barrier()

            def body(i, _):
                base = pl.multiple_of(wid * (chunks * EC) + i * EC, EC)
                pltpu.sync_copy(src_hbm.at[pl.ds(base, EC)], src_v)
                pltpu.sync_copy(dst_hbm.at[pl.ds(base, EC)], dst_v)
                pltpu.sync_copy(norm_hbm.at[pl.ds(base, EC)], norm_v)
                for t in range(EC // L):
                    sl = pl.ds(t * L, L)
                    idxb_v[sl] = (src_v[sl] + b * npad) * 128
                pltpu.async_copy(hw_hbm.at[idxb_v], rows_v, sem).wait()
                for t in range(EC // L):
                    sl = pl.ds(t * L, L)
                    rows_v[sl] = rows_v[sl] * norm_v[sl]
                pltpu.sync_copy(rows_v, acc.at[dst_v], add=True)
                return 0

            lax.fori_loop(0, chunks, body, 0)
            plsc.subcore_barrier()
            pltpu.sync_copy(acc.at[pl.ds(s * rpt, rpt)],
                            out_hbm.at[c, b, pl.ds(s * rpt, rpt)])
            plsc.subcore_barrier()

    return k


BM = 256


def _dinv_body(deg_ref, out_ref):
    d = deg_ref[0] + deg_ref[1]
    out_ref[...] = jnp.where(
        d > 0, lax.rsqrt(jnp.maximum(d, 1e-12)), 0.0)


def _dinv(degp, npad):
    r = npad // 128
    return pl.pallas_call(
        _dinv_body,
        out_shape=jax.ShapeDtypeStruct((r, 128), jnp.float32),
    )(degp.reshape(NC, r, 128)).reshape(npad)


def _mm_body(x_ref, w_ref, o_ref):
    o_ref[0] = jnp.dot(x_ref[0], w_ref[...],
                       preferred_element_type=jnp.float32)


def _mm(x, w, nb, npad):
    return pl.pallas_call(
        _mm_body,
        grid=(nb, npad // BM),
        in_specs=[
            pl.BlockSpec((1, BM, 128), lambda b, j: (b, j, 0)),
            pl.BlockSpec((128, 128), lambda b, j: (0, 0)),
        ],
        out_specs=pl.BlockSpec((1, BM, 128), lambda b, j: (b, j, 0)),
        out_shape=jax.ShapeDtypeStruct((nb, npad, 128), jnp.float32),
    )(x, w)


def _comb_mm_body(p_ref, b_ref, w_ref, o_ref):
    h = jnp.maximum(p_ref[0, 0] + p_ref[1, 0] + b_ref[...], 0.0)
    o_ref[0] = jnp.dot(h, w_ref[...], preferred_element_type=jnp.float32)


def _comb_mm(parts, bias, w, nb, npad):
    return pl.pallas_call(
        _comb_mm_body,
        grid=(nb, npad // BM),
        in_specs=[
            pl.BlockSpec((NC, 1, BM, 128), lambda b, j: (0, b, j, 0)),
            pl.BlockSpec((1, 128), lambda b, j: (0, 0)),
            pl.BlockSpec((128, 128), lambda b, j: (0, 0)),
        ],
        out_specs=pl.BlockSpec((1, BM, 128), lambda b, j: (b, j, 0)),
        out_shape=jax.ShapeDtypeStruct((nb, npad, 128), jnp.float32),
    )(parts, bias.reshape(1, 128), w)


def _final_body(p_ref, b_ref, o_ref):
    o_ref[...] = p_ref[0] + p_ref[1] + b_ref[0]


def _final(parts, b4, nb, npad):
    r = npad // 128
    return pl.pallas_call(
        _final_body,
        in_specs=[
            pl.BlockSpec(memory_space=pltpu.VMEM),
            pl.BlockSpec(memory_space=pltpu.SMEM),
        ],
        out_shape=jax.ShapeDtypeStruct((nb, r, 128), jnp.float32),
    )(parts.reshape(NC, nb, r, 128), b4).reshape(nb, npad)


def kernel(x, edge_index, edge_weight, W1, b1, W2, b2, W3, b3, W4, b4):
    nb, n, ch = x.shape
    e = edge_weight.shape[0]

    npad = _round_up(n, NS * EC)
    ep = _round_up(e + n, NT * EC)

    loop = jnp.arange(n, dtype=edge_index.dtype)
    src = jnp.concatenate([edge_index[0], loop])
    dst = jnp.concatenate([edge_index[1], loop])
    ew = jnp.concatenate([edge_weight, jnp.ones((n,), edge_weight.dtype)])
    pad = ep - (e + n)
    src = jnp.pad(src, (0, pad))
    dst = jnp.pad(dst, (0, pad))
    ew = jnp.pad(ew, (0, pad))
    xp = jnp.pad(x, ((0, 0), (0, npad - n), (0, 0)))

    degp = _deg_kernel(ep, npad)(dst, ew)
    dinv = _dinv(degp, npad)
    norm = _norm_kernel(ep)(src, dst, ew, dinv)

    spmm = _spmm_kernel(ep, npad, nb)
    hw = _mm(xp, W1, nb, npad)
    parts = spmm(hw.reshape(nb * npad, 128), src, dst, norm)
    hw = _comb_mm(parts, b1, W2, nb, npad)
    parts = spmm(hw.reshape(nb * npad, 128), src, dst, norm)
    hw = _comb_mm(parts, b2, W3, nb, npad)
    parts = spmm(hw.reshape(nb * npad, 128), src, dst, norm)
    w4p = jnp.pad(W4, ((0, 0), (0, 128 - W4.shape[1])))
    hw4 = _comb_mm(parts, b3, w4p, nb, npad)
    parts4 = _spmm1_kernel(ep, npad, nb)(
        hw4.reshape(nb * npad * 128), src, dst, norm)
    y = _final(parts4, b4, nb, npad)
    return y[:, :n]
